# Initial kernel scaffold; baseline (speedup 1.0000x reference)
import jax
import jax.numpy as jnp
from jax.experimental import pallas as pl


def kernel(inputs, input_table, position_table):
    raise NotImplementedError("write your pallas kernel here")


# speedup vs baseline: 1.2672x; 1.2672x over previous
import functools

import jax
import jax.numpy as jnp
from jax import lax
from jax.experimental import pallas as pl
from jax.experimental.pallas import tpu as pltpu
from jax.experimental.pallas import tpu_sc as plsc

_NUM_CORES = 2
_NUM_SUBCORES = 16
_NUM_WORKERS = _NUM_CORES * _NUM_SUBCORES
_CHUNK = 128


def kernel(inputs, input_table, position_table):
    batch, seqlen = inputs.shape
    vocab, embed = input_table.shape
    n = batch * seqlen
    rpw = n // _NUM_WORKERS
    n_chunks = rpw // _CHUNK
    pos_blocks = seqlen // rpw

    idx3 = inputs.reshape(_NUM_WORKERS, n_chunks, _CHUNK)
    pos3 = position_table.reshape(pos_blocks, rpw, embed)
    scat = (jnp.arange(_NUM_SUBCORES, dtype=jnp.int32)[:, None] * rpw
            + jnp.arange(rpw, dtype=jnp.int32)[None, :]
            ).reshape(_NUM_SUBCORES, n_chunks, _CHUNK)

    mesh = plsc.VectorSubcoreMesh(
        core_axis_name="c", subcore_axis_name="s",
        num_cores=_NUM_CORES, num_subcores=_NUM_SUBCORES)

    @functools.partial(
        pl.kernel,
        out_type=jax.ShapeDtypeStruct(
            (_NUM_WORKERS, rpw, embed), jnp.float32),
        mesh=mesh,
        scratch_types=[
            pltpu.VMEM((n_chunks, _CHUNK), jnp.int32),
            pltpu.VMEM((n_chunks, _CHUNK), jnp.int32),
            pltpu.VMEM((rpw, embed), jnp.float32),
            pltpu.VMEM_SHARED((_NUM_SUBCORES * rpw, embed), jnp.float32),
            pltpu.SemaphoreType.DMA,
            pltpu.SemaphoreType.DMA,
        ],
    )
    def emb_kernel(idx_hbm, tab_hbm, pos_hbm, scat_hbm, out_hbm,
                   idx_v, scat_v, rows_v, shared, sem, sem2):
        c = lax.axis_index("c")
        s = lax.axis_index("s")
        wid = s * _NUM_CORES + c
        my_shared = shared.at[pl.ds(s * rpw, rpw)]
        cp_pos = pltpu.async_copy(pos_hbm.at[wid % pos_blocks], my_shared,
                                  sem2)
        cp_idx = pltpu.async_copy(idx_hbm.at[wid], idx_v, sem)
        cp_scat = pltpu.async_copy(scat_hbm.at[s], scat_v, sem)
        cp_idx.wait()
        cp_scat.wait()
        gathers = [
            pltpu.async_copy(
                tab_hbm.at[idx_v.at[j]],
                rows_v.at[pl.ds(j * _CHUNK, _CHUNK)], sem)
            for j in range(n_chunks)
        ]
        for g in gathers:
            g.wait()
        cp_pos.wait()
        for j in range(n_chunks):
            pltpu.sync_copy(
                rows_v.at[pl.ds(j * _CHUNK, _CHUNK)],
                shared.at[scat_v.at[j]], add=True)
        pltpu.sync_copy(my_shared, out_hbm.at[wid])

    out = emb_kernel(idx3, input_table, pos3, scat)
    return out.reshape(batch, seqlen, embed)


# speedup vs baseline: 1.3628x; 1.0755x over previous
import functools

import jax
import jax.numpy as jnp
import numpy as np
from jax import lax
from jax.experimental import pallas as pl
from jax.experimental.pallas import tpu as pltpu
from jax.experimental.pallas import tpu_sc as plsc

_NUM_CORES = 2
_NUM_SUBCORES = 16
_NUM_WORKERS = _NUM_CORES * _NUM_SUBCORES
_CHUNK = 128


def kernel(inputs, input_table, position_table):
    batch, seqlen = inputs.shape
    vocab, embed = input_table.shape
    n = batch * seqlen
    rpw = n // _NUM_WORKERS
    n_chunks = rpw // _CHUNK
    wpb = seqlen // rpw

    scat = jnp.asarray(
        (np.arange(_NUM_SUBCORES, dtype=np.int32)[:, None] * rpw
         + np.arange(rpw, dtype=np.int32)[None, :]
         ).reshape(_NUM_SUBCORES, n_chunks, _CHUNK))

    mesh = plsc.VectorSubcoreMesh(
        core_axis_name="c", subcore_axis_name="s",
        num_cores=_NUM_CORES, num_subcores=_NUM_SUBCORES)

    @functools.partial(
        pl.kernel,
        out_type=jax.ShapeDtypeStruct((batch, seqlen, embed), jnp.float32),
        mesh=mesh,
        scratch_types=[
            pltpu.VMEM((rpw,), jnp.int32),
            pltpu.VMEM((n_chunks, _CHUNK), jnp.int32),
            pltpu.VMEM((rpw, embed), jnp.float32),
            pltpu.VMEM_SHARED((_NUM_SUBCORES * rpw, embed), jnp.float32),
            pltpu.SemaphoreType.DMA,
            pltpu.SemaphoreType.DMA,
            pltpu.SemaphoreType.DMA,
        ],
    )
    def emb_kernel(idx_hbm, tab_hbm, pos_hbm, scat_hbm, out_hbm,
                   idx_v, scat_v, rows_v, shared, sem, sem2, sem3):
        c = lax.axis_index("c")
        s = lax.axis_index("s")
        wid = s * _NUM_CORES + c
        b = wid // wpb
        col = (wid % wpb) * rpw
        my_shared = shared.at[pl.ds(s * rpw, rpw)]
        cp_pos = pltpu.async_copy(pos_hbm.at[pl.ds(col, rpw)], my_shared,
                                  sem2)
        cp_idx = pltpu.async_copy(idx_hbm.at[b, pl.ds(col, rpw)], idx_v, sem)
        cp_scat = pltpu.async_copy(scat_hbm.at[s], scat_v, sem)
        cp_idx.wait()
        cp_scat.wait()
        gathers = [
            pltpu.async_copy(
                tab_hbm.at[idx_v.at[pl.ds(j * _CHUNK, _CHUNK)]],
                rows_v.at[pl.ds(j * _CHUNK, _CHUNK)], sem)
            for j in range(n_chunks)
        ]
        cp_pos.wait()
        outs = []
        for j in range(n_chunks):
            gathers[j].wait()
            pltpu.sync_copy(
                rows_v.at[pl.ds(j * _CHUNK, _CHUNK)],
                shared.at[scat_v.at[j]], add=True)
            outs.append(pltpu.async_copy(
                shared.at[pl.ds(s * rpw + j * _CHUNK, _CHUNK)],
                out_hbm.at[b, pl.ds(col + j * _CHUNK, _CHUNK)], sem3))
        for o in outs:
            o.wait()

    return emb_kernel(inputs, input_table, position_table, scat)


# speedup vs baseline: 1.3638x; 1.0007x over previous
import functools

import jax
import jax.numpy as jnp
from jax import lax
from jax.experimental import pallas as pl
from jax.experimental.pallas import tpu as pltpu
from jax.experimental.pallas import tpu_sc as plsc

_NUM_CORES = 2
_NUM_SUBCORES = 16
_NUM_WORKERS = _NUM_CORES * _NUM_SUBCORES
_CHUNK = 128


def kernel(inputs, input_table, position_table):
    batch, seqlen = inputs.shape
    vocab, embed = input_table.shape
    n = batch * seqlen
    rpw = n // _NUM_WORKERS
    n_chunks = rpw // _CHUNK
    wpb = seqlen // rpw

    mesh = plsc.VectorSubcoreMesh(
        core_axis_name="c", subcore_axis_name="s",
        num_cores=_NUM_CORES, num_subcores=_NUM_SUBCORES)

    @functools.partial(
        pl.kernel,
        out_type=jax.ShapeDtypeStruct((batch, seqlen, embed), jnp.float32),
        mesh=mesh,
        scratch_types=[
            pltpu.VMEM((rpw,), jnp.int32),
            pltpu.VMEM((n_chunks, _CHUNK), jnp.int32),
            pltpu.VMEM((rpw, embed), jnp.float32),
            pltpu.VMEM_SHARED((_NUM_SUBCORES * rpw, embed), jnp.float32),
            pltpu.SemaphoreType.DMA,
            pltpu.SemaphoreType.DMA,
            pltpu.SemaphoreType.DMA,
        ],
    )
    def emb_kernel(idx_hbm, tab_hbm, pos_hbm, out_hbm,
                   idx_v, scat_v, rows_v, shared, sem, sem2, sem3):
        c = lax.axis_index("c")
        s = lax.axis_index("s")
        wid = s * _NUM_CORES + c
        b = wid // wpb
        col = (wid % wpb) * rpw
        my_shared = shared.at[pl.ds(s * rpw, rpw)]
        cp_pos = pltpu.async_copy(pos_hbm.at[pl.ds(col, rpw)], my_shared,
                                  sem2)
        cp_idx = pltpu.async_copy(idx_hbm.at[b, pl.ds(col, rpw)], idx_v, sem)
        lanes = lax.iota(jnp.int32, 16)
        for j in range(n_chunks):
            for k in range(_CHUNK // 16):
                scat_v[j, pl.ds(k * 16, 16)] = lanes + (
                    s * rpw + j * _CHUNK + k * 16)
        cp_idx.wait()
        gathers = [
            pltpu.async_copy(
                tab_hbm.at[idx_v.at[pl.ds(j * _CHUNK, _CHUNK)]],
                rows_v.at[pl.ds(j * _CHUNK, _CHUNK)], sem)
            for j in range(n_chunks)
        ]
        cp_pos.wait()
        outs = []
        for j in range(n_chunks):
            gathers[j].wait()
            pltpu.sync_copy(
                rows_v.at[pl.ds(j * _CHUNK, _CHUNK)],
                shared.at[scat_v.at[j]], add=True)
            outs.append(pltpu.async_copy(
                shared.at[pl.ds(s * rpw + j * _CHUNK, _CHUNK)],
                out_hbm.at[b, pl.ds(col + j * _CHUNK, _CHUNK)], sem3))
        for o in outs:
            o.wait()

    return emb_kernel(inputs, input_table, position_table)


# speedup vs baseline: 1.3747x; 1.0080x over previous
import functools

import jax
import jax.numpy as jnp
from jax import lax
from jax.experimental import pallas as pl
from jax.experimental.pallas import tpu as pltpu
from jax.experimental.pallas import tpu_sc as plsc

_NUM_CORES = 2
_NUM_SUBCORES = 16
_NUM_WORKERS = _NUM_CORES * _NUM_SUBCORES
_CHUNK = 64


def kernel(inputs, input_table, position_table):
    batch, seqlen = inputs.shape
    vocab, embed = input_table.shape
    n = batch * seqlen
    rpw = n // _NUM_WORKERS
    n_chunks = rpw // _CHUNK
    wpb = seqlen // rpw

    mesh = plsc.VectorSubcoreMesh(
        core_axis_name="c", subcore_axis_name="s",
        num_cores=_NUM_CORES, num_subcores=_NUM_SUBCORES)

    @functools.partial(
        pl.kernel,
        out_type=jax.ShapeDtypeStruct((batch, seqlen, embed), jnp.float32),
        mesh=mesh,
        scratch_types=[
            pltpu.VMEM((rpw,), jnp.int32),
            pltpu.VMEM((n_chunks, _CHUNK), jnp.int32),
            pltpu.VMEM((rpw, embed), jnp.float32),
            pltpu.VMEM_SHARED((_NUM_SUBCORES * rpw, embed), jnp.float32),
            pltpu.SemaphoreType.DMA,
            pltpu.SemaphoreType.DMA,
            pltpu.SemaphoreType.DMA,
        ],
    )
    def emb_kernel(idx_hbm, tab_hbm, pos_hbm, out_hbm,
                   idx_v, scat_v, rows_v, shared, sem, sem2, sem3):
        c = lax.axis_index("c")
        s = lax.axis_index("s")
        wid = s * _NUM_CORES + c
        b = wid // wpb
        col = (wid % wpb) * rpw
        my_shared = shared.at[pl.ds(s * rpw, rpw)]
        cp_pos = pltpu.async_copy(pos_hbm.at[pl.ds(col, rpw)], my_shared,
                                  sem2)
        cp_idx = pltpu.async_copy(idx_hbm.at[b, pl.ds(col, rpw)], idx_v, sem)
        lanes = lax.iota(jnp.int32, 16)
        for j in range(n_chunks):
            for k in range(_CHUNK // 16):
                scat_v[j, pl.ds(k * 16, 16)] = lanes + (
                    s * rpw + j * _CHUNK + k * 16)
        cp_idx.wait()
        gathers = [
            pltpu.async_copy(
                tab_hbm.at[idx_v.at[pl.ds(j * _CHUNK, _CHUNK)]],
                rows_v.at[pl.ds(j * _CHUNK, _CHUNK)], sem)
            for j in range(n_chunks)
        ]
        cp_pos.wait()
        outs = []
        for j in range(n_chunks):
            gathers[j].wait()
            pltpu.sync_copy(
                rows_v.at[pl.ds(j * _CHUNK, _CHUNK)],
                shared.at[scat_v.at[j]], add=True)
            outs.append(pltpu.async_copy(
                shared.at[pl.ds(s * rpw + j * _CHUNK, _CHUNK)],
                out_hbm.at[b, pl.ds(col + j * _CHUNK, _CHUNK)], sem3))
        for o in outs:
            o.wait()

    return emb_kernel(inputs, input_table, position_table)
